# baseline (speedup 1.0000x reference)
import functools

import jax
import jax.numpy as jnp
from jax import lax
from jax.experimental import pallas as pl
from jax.experimental.pallas import tpu as pltpu
from jax.experimental.pallas import tpu_sc as plsc

N = 10000
NNZ = 320000
NC = 2
NS = 16
K = 100
NCH = NNZ // (NC * NS * K)
NHV = 2
NH = NCH // NHV
RA, RB = 624, N - 15 * 624

_mesh = plsc.VectorSubcoreMesh(core_axis_name="c", subcore_axis_name="s")


def _sc_segment_sum(table, src_idx, dst_idx, zeros_pad):
    D = table.shape[1]

    @functools.partial(
        pl.kernel,
        out_type=jax.ShapeDtypeStruct((NC * N, D), jnp.float32),
        mesh=_mesh,
        scratch_types=[
            pltpu.VMEM_SHARED((N, D), jnp.float32),
            pltpu.VMEM((NH, K), jnp.int32),
            pltpu.VMEM((NH, K), jnp.int32),
            pltpu.VMEM((K, D), jnp.float32),
            pltpu.VMEM((K, D), jnp.float32),
            pltpu.SemaphoreType.DMA,
            pltpu.SemaphoreType.DMA,
            pltpu.SemaphoreType.DMA,
            pltpu.SemaphoreType.DMA,
        ],
    )
    def run(t_hbm, src_hbm, dst_hbm, z_hbm, out,
            acc, src_v, dst_v, rb0, rb1, g0, g1, s0, s1):
        c = lax.axis_index("c")
        s = lax.axis_index("s")

        @pl.when(s < 15)
        def _():
            pltpu.sync_copy(z_hbm.at[pl.ds(s * RA, RA)],
                            acc.at[pl.ds(s * RA, RA)])

        @pl.when(s == 15)
        def _():
            pltpu.sync_copy(z_hbm.at[pl.ds(15 * RA, RB)],
                            acc.at[pl.ds(15 * RA, RB)])

        plsc.subcore_barrier()

        for h in range(NHV):
            pltpu.sync_copy(src_hbm.at[(c * NS + s) * NHV + h], src_v)
            pltpu.sync_copy(dst_hbm.at[(c * NS + s) * NHV + h], dst_v)

            pltpu.async_copy(t_hbm.at[src_v.at[0]], rb0, g0)
            pltpu.async_copy(t_hbm.at[src_v.at[1]], rb1, g1)

            pltpu.make_async_copy(t_hbm.at[src_v.at[0]], rb0, g0).wait()
            pltpu.async_copy(rb0, acc.at[dst_v.at[0]], s0, add=True)
            pltpu.make_async_copy(t_hbm.at[src_v.at[1]], rb1, g1).wait()
            pltpu.async_copy(rb1, acc.at[dst_v.at[1]], s1, add=True)

            @pl.loop(2, NH - 2, step=2)
            def _(i):
                pltpu.make_async_copy(rb0, acc.at[dst_v.at[i - 2]], s0).wait()
                pltpu.async_copy(t_hbm.at[src_v.at[i]], rb0, g0)
                pltpu.make_async_copy(rb1, acc.at[dst_v.at[i - 1]], s1).wait()
                pltpu.async_copy(t_hbm.at[src_v.at[i + 1]], rb1, g1)
                pltpu.make_async_copy(t_hbm.at[src_v.at[i]], rb0, g0).wait()
                pltpu.async_copy(rb0, acc.at[dst_v.at[i]], s0, add=True)
                pltpu.make_async_copy(t_hbm.at[src_v.at[i + 1]], rb1, g1).wait()
                pltpu.async_copy(rb1, acc.at[dst_v.at[i + 1]], s1, add=True)

            pltpu.make_async_copy(rb0, acc.at[dst_v.at[NH - 4]], s0).wait()
            pltpu.async_copy(t_hbm.at[src_v.at[NH - 2]], rb0, g0)
            pltpu.make_async_copy(rb1, acc.at[dst_v.at[NH - 3]], s1).wait()
            pltpu.async_copy(t_hbm.at[src_v.at[NH - 1]], rb1, g1)
            pltpu.make_async_copy(t_hbm.at[src_v.at[NH - 2]], rb0, g0).wait()
            pltpu.async_copy(rb0, acc.at[dst_v.at[NH - 2]], s0, add=True)
            pltpu.make_async_copy(t_hbm.at[src_v.at[NH - 1]], rb1, g1).wait()
            pltpu.async_copy(rb1, acc.at[dst_v.at[NH - 1]], s1, add=True)
            pltpu.make_async_copy(rb0, acc.at[dst_v.at[NH - 2]], s0).wait()
            pltpu.make_async_copy(rb1, acc.at[dst_v.at[NH - 1]], s1).wait()

        plsc.subcore_barrier()

        @pl.when(s < 15)
        def _():
            pltpu.sync_copy(acc.at[pl.ds(s * RA, RA)],
                            out.at[pl.ds(c * N + s * RA, RA)])

        @pl.when(s == 15)
        def _():
            pltpu.sync_copy(acc.at[pl.ds(15 * RA, RB)],
                            out.at[pl.ds(c * N + 15 * RA, RB)])

    return run(table, src_idx, dst_idx, zeros_pad)


def _sc_degrees(eidx, vidx, ones_k, zeros_pad):
    shp = jax.ShapeDtypeStruct((NC * N, 128), jnp.float32)

    @functools.partial(
        pl.kernel,
        out_type=(shp, shp),
        mesh=_mesh,
        scratch_types=[
            pltpu.VMEM_SHARED((N, 128), jnp.float32),
            pltpu.VMEM((NCH, K), jnp.int32),
            pltpu.VMEM((K, 128), jnp.float32),
        ],
    )
    def run(e_hbm, v_hbm, ones_hbm, z_hbm, cnt_e, cnt_v,
            acc, idx_v, ones_v):
        c = lax.axis_index("c")
        s = lax.axis_index("s")
        pltpu.sync_copy(ones_hbm, ones_v)
        for idx_hbm, out in ((e_hbm, cnt_e), (v_hbm, cnt_v)):
            pltpu.sync_copy(idx_hbm.at[c * NS + s], idx_v)

            @pl.when(s < 15)
            def _():
                pltpu.sync_copy(z_hbm.at[pl.ds(s * RA, RA)],
                                acc.at[pl.ds(s * RA, RA)])

            @pl.when(s == 15)
            def _():
                pltpu.sync_copy(z_hbm.at[pl.ds(15 * RA, RB)],
                                acc.at[pl.ds(15 * RA, RB)])

            plsc.subcore_barrier()

            @pl.loop(0, NCH)
            def _(j):
                pltpu.sync_copy(ones_v, acc.at[idx_v.at[j]], add=True)

            plsc.subcore_barrier()

            @pl.when(s < 15)
            def _():
                pltpu.sync_copy(acc.at[pl.ds(s * RA, RA)],
                                out.at[pl.ds(c * N + s * RA, RA)])

            @pl.when(s == 15)
            def _():
                pltpu.sync_copy(acc.at[pl.ds(15 * RA, RB)],
                                out.at[pl.ds(c * N + 15 * RA, RB)])

            plsc.subcore_barrier()

    return run(eidx, vidx, ones_k, zeros_pad)


_BLK = 1000


def _rowspec(d):
    return pl.BlockSpec((_BLK, d), lambda i: (i, 0))


def _dot(a, b):
    return lax.dot_general(a, b, (((1,), (0,)), ((), ())),
                           preferred_element_type=jnp.float32,
                           precision=lax.Precision.HIGHEST)


def _tc_matmul(X, W, b, d_pad):
    n, d_in = X.shape
    d_out = W.shape[1]

    def body(x_ref, w_ref, b_ref, o_ref):
        y = _dot(x_ref[...], w_ref[...]) + b_ref[...]
        if d_pad > d_out:
            y = jnp.concatenate(
                [y, jnp.zeros((_BLK, d_pad - d_out), jnp.float32)], axis=1)
        o_ref[...] = y

    return pl.pallas_call(
        body,
        grid=(n // _BLK,),
        in_specs=[_rowspec(d_in),
                  pl.BlockSpec((d_in, d_out), lambda i: (0, 0)),
                  pl.BlockSpec((1, d_out), lambda i: (0, 0))],
        out_specs=_rowspec(d_pad),
        out_shape=jax.ShapeDtypeStruct((n, d_pad), jnp.float32),
    )(X, W, b.reshape(1, -1))


def _tc_inv(cnt):
    def body(c0_ref, c1_ref, o_ref):
        t = jnp.maximum(c0_ref[...][:, 0:8] + c1_ref[...][:, 0:8], 1.0)
        o_ref[...] = 1.0 / t

    return pl.pallas_call(
        body,
        grid=(N // _BLK,),
        in_specs=[_rowspec(128), _rowspec(128)],
        out_specs=_rowspec(8),
        out_shape=jax.ShapeDtypeStruct((N, 8), jnp.float32),
    )(cnt[:N], cnt[N:])


def _tc_scale(ssum, inv, relu, d_out=None):
    d = ssum.shape[1]
    d_out = d_out or d

    def body(s0_ref, s1_ref, i_ref, o_ref):
        r = (s0_ref[...][:, :d_out] + s1_ref[...][:, :d_out]) * i_ref[...][:, 0:1]
        if relu:
            r = jnp.maximum(r, 0.0)
        o_ref[...] = r

    return pl.pallas_call(
        body,
        grid=(N // _BLK,),
        in_specs=[_rowspec(d), _rowspec(d), _rowspec(8)],
        out_specs=_rowspec(d_out),
        out_shape=jax.ShapeDtypeStruct((N, d_out), jnp.float32),
    )(ssum[:N], ssum[N:], inv)


def _tc_boundary(ssum, inv, x_res, W, b, d_pad):
    d = ssum.shape[1]
    d_out = W.shape[1]
    with_res = x_res is not None

    def body(*refs):
        if with_res:
            s0_ref, s1_ref, i_ref, xr_ref, w_ref, b_ref, z_ref, y_ref = refs
        else:
            s0_ref, s1_ref, i_ref, w_ref, b_ref, z_ref, y_ref = refs
        z = jnp.maximum((s0_ref[...] + s1_ref[...]) * i_ref[...][:, 0:1], 0.0)
        if with_res:
            z = z + xr_ref[...]
        z_ref[...] = z
        y = _dot(z, w_ref[...]) + b_ref[...]
        if d_pad > d_out:
            y = jnp.concatenate(
                [y, jnp.zeros((_BLK, d_pad - d_out), jnp.float32)], axis=1)
        y_ref[...] = y

    in_specs = [_rowspec(d), _rowspec(d), _rowspec(8)]
    args = [ssum[:N], ssum[N:], inv]
    if with_res:
        in_specs.append(_rowspec(d))
        args.append(x_res)
    in_specs += [pl.BlockSpec((d, d_out), lambda i: (0, 0)),
                 pl.BlockSpec((1, d_out), lambda i: (0, 0))]
    args += [W, b.reshape(1, -1)]

    return pl.pallas_call(
        body,
        grid=(N // _BLK,),
        in_specs=in_specs,
        out_specs=[_rowspec(d), _rowspec(d_pad)],
        out_shape=[jax.ShapeDtypeStruct((N, d), jnp.float32),
                   jax.ShapeDtypeStruct((N, d_pad), jnp.float32)],
    )(*args)


def kernel(X, edge_index, W1, b1, W2, b2, W3, b3):
    vids = edge_index[0].reshape(NC * NS * NHV, NH, K)
    eids = edge_index[1].reshape(NC * NS * NHV, NH, K)
    vids_d = edge_index[0].reshape(NC * NS, NCH, K)
    eids_d = edge_index[1].reshape(NC * NS, NCH, K)
    zeros_pad = jnp.zeros((N, 128), jnp.float32)
    zeros_cnt = jnp.zeros((N, 128), jnp.float32)
    ones_k = jnp.ones((K, 128), jnp.float32)

    cnt_e, cnt_v = _sc_degrees(eids_d, vids_d, ones_k, zeros_cnt)
    inv_e = _tc_inv(cnt_e)
    inv_v = _tc_inv(cnt_v)

    y1 = _tc_matmul(X, W1, b1, 128)
    s = _sc_segment_sum(y1, vids, eids, zeros_pad)
    e1 = _tc_scale(s, inv_e, relu=False)
    s = _sc_segment_sum(e1, eids, vids, zeros_pad)
    x1, y2 = _tc_boundary(s, inv_v, None, W2, b2, 128)

    s = _sc_segment_sum(y2, vids, eids, zeros_pad)
    e2 = _tc_scale(s, inv_e, relu=False)
    s = _sc_segment_sum(e2, eids, vids, zeros_pad)
    _, y3 = _tc_boundary(s, inv_v, x1, W3, b3, 128)

    s = _sc_segment_sum(y3, vids, eids, zeros_pad)
    e3 = _tc_scale(s, inv_e, relu=False)
    s = _sc_segment_sum(e3, eids, vids, zeros_pad)
    x3 = _tc_scale(s, inv_v, relu=True, d_out=64)
    return x3


# speedup vs baseline: 1.0030x; 1.0030x over previous
import functools

import jax
import jax.numpy as jnp
from jax import lax
from jax.experimental import pallas as pl
from jax.experimental.pallas import tpu as pltpu
from jax.experimental.pallas import tpu_sc as plsc

N = 10000
NNZ = 320000
NC = 2
NS = 16
K = 50
NCH = NNZ // (NC * NS * K)
NHV = 2
NH = NCH // NHV
RA, RB = 624, N - 15 * 624

_mesh = plsc.VectorSubcoreMesh(core_axis_name="c", subcore_axis_name="s")


def _sc_segment_sum(table, src_idx, dst_idx, zeros_pad):
    D = table.shape[1]

    @functools.partial(
        pl.kernel,
        out_type=jax.ShapeDtypeStruct((NC * N, D), jnp.float32),
        mesh=_mesh,
        scratch_types=[
            pltpu.VMEM_SHARED((N, D), jnp.float32),
            pltpu.VMEM((NH, K), jnp.int32),
            pltpu.VMEM((NH, K), jnp.int32),
            pltpu.VMEM((K, D), jnp.float32),
            pltpu.VMEM((K, D), jnp.float32),
            pltpu.SemaphoreType.DMA,
            pltpu.SemaphoreType.DMA,
        ],
    )
    def run(t_hbm, src_hbm, dst_hbm, z_hbm, out,
            acc, src_v, dst_v, rb0, rb1, g0, g1):
        c = lax.axis_index("c")
        s = lax.axis_index("s")

        @pl.when(s < 15)
        def _():
            pltpu.sync_copy(z_hbm.at[pl.ds(s * RA, RA)],
                            acc.at[pl.ds(s * RA, RA)])

        @pl.when(s == 15)
        def _():
            pltpu.sync_copy(z_hbm.at[pl.ds(15 * RA, RB)],
                            acc.at[pl.ds(15 * RA, RB)])

        plsc.subcore_barrier()

        for h in range(NHV):
            pltpu.sync_copy(src_hbm.at[(c * NS + s) * NHV + h], src_v)
            pltpu.sync_copy(dst_hbm.at[(c * NS + s) * NHV + h], dst_v)

            pltpu.async_copy(t_hbm.at[src_v.at[0]], rb0, g0)
            pltpu.async_copy(t_hbm.at[src_v.at[1]], rb1, g1)

            @pl.loop(0, NH - 4, step=2)
            def _(i):
                pltpu.make_async_copy(t_hbm.at[src_v.at[i]], rb0, g0).wait()
                pltpu.sync_copy(rb0, acc.at[dst_v.at[i]], add=True)
                pltpu.async_copy(t_hbm.at[src_v.at[i + 2]], rb0, g0)
                pltpu.make_async_copy(t_hbm.at[src_v.at[i + 1]], rb1, g1).wait()
                pltpu.sync_copy(rb1, acc.at[dst_v.at[i + 1]], add=True)
                pltpu.async_copy(t_hbm.at[src_v.at[i + 3]], rb1, g1)

            pltpu.make_async_copy(t_hbm.at[src_v.at[NH - 4]], rb0, g0).wait()
            pltpu.sync_copy(rb0, acc.at[dst_v.at[NH - 4]], add=True)
            pltpu.async_copy(t_hbm.at[src_v.at[NH - 2]], rb0, g0)
            pltpu.make_async_copy(t_hbm.at[src_v.at[NH - 3]], rb1, g1).wait()
            pltpu.sync_copy(rb1, acc.at[dst_v.at[NH - 3]], add=True)
            pltpu.async_copy(t_hbm.at[src_v.at[NH - 1]], rb1, g1)
            pltpu.make_async_copy(t_hbm.at[src_v.at[NH - 2]], rb0, g0).wait()
            pltpu.sync_copy(rb0, acc.at[dst_v.at[NH - 2]], add=True)
            pltpu.make_async_copy(t_hbm.at[src_v.at[NH - 1]], rb1, g1).wait()
            pltpu.sync_copy(rb1, acc.at[dst_v.at[NH - 1]], add=True)

        plsc.subcore_barrier()

        @pl.when(s < 15)
        def _():
            pltpu.sync_copy(acc.at[pl.ds(s * RA, RA)],
                            out.at[pl.ds(c * N + s * RA, RA)])

        @pl.when(s == 15)
        def _():
            pltpu.sync_copy(acc.at[pl.ds(15 * RA, RB)],
                            out.at[pl.ds(c * N + 15 * RA, RB)])

    return run(table, src_idx, dst_idx, zeros_pad)


def _sc_degrees(eidx, vidx, ones_k, zeros_pad):
    shp = jax.ShapeDtypeStruct((NC * N, 128), jnp.float32)

    @functools.partial(
        pl.kernel,
        out_type=(shp, shp),
        mesh=_mesh,
        scratch_types=[
            pltpu.VMEM_SHARED((N, 128), jnp.float32),
            pltpu.VMEM((NCH, K), jnp.int32),
            pltpu.VMEM((K, 128), jnp.float32),
        ],
    )
    def run(e_hbm, v_hbm, ones_hbm, z_hbm, cnt_e, cnt_v,
            acc, idx_v, ones_v):
        c = lax.axis_index("c")
        s = lax.axis_index("s")
        pltpu.sync_copy(ones_hbm, ones_v)
        for idx_hbm, out in ((e_hbm, cnt_e), (v_hbm, cnt_v)):
            pltpu.sync_copy(idx_hbm.at[c * NS + s], idx_v)

            @pl.when(s < 15)
            def _():
                pltpu.sync_copy(z_hbm.at[pl.ds(s * RA, RA)],
                                acc.at[pl.ds(s * RA, RA)])

            @pl.when(s == 15)
            def _():
                pltpu.sync_copy(z_hbm.at[pl.ds(15 * RA, RB)],
                                acc.at[pl.ds(15 * RA, RB)])

            plsc.subcore_barrier()

            @pl.loop(0, NCH)
            def _(j):
                pltpu.sync_copy(ones_v, acc.at[idx_v.at[j]], add=True)

            plsc.subcore_barrier()

            @pl.when(s < 15)
            def _():
                pltpu.sync_copy(acc.at[pl.ds(s * RA, RA)],
                                out.at[pl.ds(c * N + s * RA, RA)])

            @pl.when(s == 15)
            def _():
                pltpu.sync_copy(acc.at[pl.ds(15 * RA, RB)],
                                out.at[pl.ds(c * N + 15 * RA, RB)])

            plsc.subcore_barrier()

    return run(eidx, vidx, ones_k, zeros_pad)


_BLK = 1000


def _rowspec(d):
    return pl.BlockSpec((_BLK, d), lambda i: (i, 0))


def _dot(a, b):
    return lax.dot_general(a, b, (((1,), (0,)), ((), ())),
                           preferred_element_type=jnp.float32,
                           precision=lax.Precision.HIGHEST)


def _tc_matmul(X, W, b, d_pad):
    n, d_in = X.shape
    d_out = W.shape[1]

    def body(x_ref, w_ref, b_ref, o_ref):
        y = _dot(x_ref[...], w_ref[...]) + b_ref[...]
        if d_pad > d_out:
            y = jnp.concatenate(
                [y, jnp.zeros((_BLK, d_pad - d_out), jnp.float32)], axis=1)
        o_ref[...] = y

    return pl.pallas_call(
        body,
        grid=(n // _BLK,),
        in_specs=[_rowspec(d_in),
                  pl.BlockSpec((d_in, d_out), lambda i: (0, 0)),
                  pl.BlockSpec((1, d_out), lambda i: (0, 0))],
        out_specs=_rowspec(d_pad),
        out_shape=jax.ShapeDtypeStruct((n, d_pad), jnp.float32),
    )(X, W, b.reshape(1, -1))


def _tc_inv(cnt):
    def body(c0_ref, c1_ref, o_ref):
        t = jnp.maximum(c0_ref[...][:, 0:8] + c1_ref[...][:, 0:8], 1.0)
        o_ref[...] = 1.0 / t

    return pl.pallas_call(
        body,
        grid=(N // _BLK,),
        in_specs=[_rowspec(128), _rowspec(128)],
        out_specs=_rowspec(8),
        out_shape=jax.ShapeDtypeStruct((N, 8), jnp.float32),
    )(cnt[:N], cnt[N:])


def _tc_scale(ssum, inv, relu, d_out=None):
    d = ssum.shape[1]
    d_out = d_out or d

    def body(s0_ref, s1_ref, i_ref, o_ref):
        r = (s0_ref[...][:, :d_out] + s1_ref[...][:, :d_out]) * i_ref[...][:, 0:1]
        if relu:
            r = jnp.maximum(r, 0.0)
        o_ref[...] = r

    return pl.pallas_call(
        body,
        grid=(N // _BLK,),
        in_specs=[_rowspec(d), _rowspec(d), _rowspec(8)],
        out_specs=_rowspec(d_out),
        out_shape=jax.ShapeDtypeStruct((N, d_out), jnp.float32),
    )(ssum[:N], ssum[N:], inv)


def _tc_boundary(ssum, inv, x_res, W, b, d_pad):
    d = ssum.shape[1]
    d_out = W.shape[1]
    with_res = x_res is not None

    def body(*refs):
        if with_res:
            s0_ref, s1_ref, i_ref, xr_ref, w_ref, b_ref, z_ref, y_ref = refs
        else:
            s0_ref, s1_ref, i_ref, w_ref, b_ref, z_ref, y_ref = refs
        z = jnp.maximum((s0_ref[...] + s1_ref[...]) * i_ref[...][:, 0:1], 0.0)
        if with_res:
            z = z + xr_ref[...]
        z_ref[...] = z
        y = _dot(z, w_ref[...]) + b_ref[...]
        if d_pad > d_out:
            y = jnp.concatenate(
                [y, jnp.zeros((_BLK, d_pad - d_out), jnp.float32)], axis=1)
        y_ref[...] = y

    in_specs = [_rowspec(d), _rowspec(d), _rowspec(8)]
    args = [ssum[:N], ssum[N:], inv]
    if with_res:
        in_specs.append(_rowspec(d))
        args.append(x_res)
    in_specs += [pl.BlockSpec((d, d_out), lambda i: (0, 0)),
                 pl.BlockSpec((1, d_out), lambda i: (0, 0))]
    args += [W, b.reshape(1, -1)]

    return pl.pallas_call(
        body,
        grid=(N // _BLK,),
        in_specs=in_specs,
        out_specs=[_rowspec(d), _rowspec(d_pad)],
        out_shape=[jax.ShapeDtypeStruct((N, d), jnp.float32),
                   jax.ShapeDtypeStruct((N, d_pad), jnp.float32)],
    )(*args)


def kernel(X, edge_index, W1, b1, W2, b2, W3, b3):
    vids = edge_index[0].reshape(NC * NS * NHV, NH, K)
    eids = edge_index[1].reshape(NC * NS * NHV, NH, K)
    vids_d = edge_index[0].reshape(NC * NS, NCH, K)
    eids_d = edge_index[1].reshape(NC * NS, NCH, K)
    zeros_pad = jnp.zeros((N, 128), jnp.float32)
    zeros_cnt = jnp.zeros((N, 128), jnp.float32)
    ones_k = jnp.ones((K, 128), jnp.float32)

    cnt_e, cnt_v = _sc_degrees(eids_d, vids_d, ones_k, zeros_cnt)
    inv_e = _tc_inv(cnt_e)
    inv_v = _tc_inv(cnt_v)

    y1 = _tc_matmul(X, W1, b1, 128)
    s = _sc_segment_sum(y1, vids, eids, zeros_pad)
    e1 = _tc_scale(s, inv_e, relu=False)
    s = _sc_segment_sum(e1, eids, vids, zeros_pad)
    x1, y2 = _tc_boundary(s, inv_v, None, W2, b2, 128)

    s = _sc_segment_sum(y2, vids, eids, zeros_pad)
    e2 = _tc_scale(s, inv_e, relu=False)
    s = _sc_segment_sum(e2, eids, vids, zeros_pad)
    _, y3 = _tc_boundary(s, inv_v, x1, W3, b3, 128)

    s = _sc_segment_sum(y3, vids, eids, zeros_pad)
    e3 = _tc_scale(s, inv_e, relu=False)
    s = _sc_segment_sum(e3, eids, vids, zeros_pad)
    x3 = _tc_scale(s, inv_v, relu=True, d_out=64)
    return x3


# speedup vs baseline: 1.2454x; 1.2417x over previous
import functools

import jax
import jax.numpy as jnp
from jax import lax
from jax.experimental import pallas as pl
from jax.experimental.pallas import tpu as pltpu
from jax.experimental.pallas import tpu_sc as plsc

N = 10000
NNZ = 320000
NC = 2
NS = 16
K = 125
NCH = NNZ // (NC * NS * K)
NHV = 2
NH = NCH // NHV
RA, RB = 624, N - 15 * 624

_mesh = plsc.VectorSubcoreMesh(core_axis_name="c", subcore_axis_name="s")


def _sc_segment_sum(table, src_idx, dst_idx, zeros_pad):
    D = table.shape[1]

    @functools.partial(
        pl.kernel,
        out_type=jax.ShapeDtypeStruct((NC * N, D), jnp.float32),
        mesh=_mesh,
        scratch_types=[
            pltpu.VMEM_SHARED((N, D), jnp.float32),
            pltpu.VMEM((NH, K), jnp.int32),
            pltpu.VMEM((NH, K), jnp.int32),
            pltpu.VMEM((K, D), jnp.float32),
            pltpu.VMEM((K, D), jnp.float32),
            pltpu.SemaphoreType.DMA,
            pltpu.SemaphoreType.DMA,
        ],
    )
    def run(t_hbm, src_hbm, dst_hbm, z_hbm, out,
            acc, src_v, dst_v, rb0, rb1, g0, g1):
        c = lax.axis_index("c")
        s = lax.axis_index("s")

        @pl.when(s < 15)
        def _():
            pltpu.sync_copy(z_hbm.at[pl.ds(s * RA, RA)],
                            acc.at[pl.ds(s * RA, RA)])

        @pl.when(s == 15)
        def _():
            pltpu.sync_copy(z_hbm.at[pl.ds(15 * RA, RB)],
                            acc.at[pl.ds(15 * RA, RB)])

        plsc.subcore_barrier()

        for h in range(NHV):
            pltpu.sync_copy(src_hbm.at[(c * NS + s) * NHV + h], src_v)
            pltpu.sync_copy(dst_hbm.at[(c * NS + s) * NHV + h], dst_v)

            pltpu.async_copy(t_hbm.at[src_v.at[0]], rb0, g0)
            pltpu.async_copy(t_hbm.at[src_v.at[1]], rb1, g1)

            @pl.loop(0, NH - 4, step=2)
            def _(i):
                pltpu.make_async_copy(t_hbm.at[src_v.at[i]], rb0, g0).wait()
                pltpu.sync_copy(rb0, acc.at[dst_v.at[i]], add=True)
                pltpu.async_copy(t_hbm.at[src_v.at[i + 2]], rb0, g0)
                pltpu.make_async_copy(t_hbm.at[src_v.at[i + 1]], rb1, g1).wait()
                pltpu.sync_copy(rb1, acc.at[dst_v.at[i + 1]], add=True)
                pltpu.async_copy(t_hbm.at[src_v.at[i + 3]], rb1, g1)

            pltpu.make_async_copy(t_hbm.at[src_v.at[NH - 4]], rb0, g0).wait()
            pltpu.sync_copy(rb0, acc.at[dst_v.at[NH - 4]], add=True)
            pltpu.async_copy(t_hbm.at[src_v.at[NH - 2]], rb0, g0)
            pltpu.make_async_copy(t_hbm.at[src_v.at[NH - 3]], rb1, g1).wait()
            pltpu.sync_copy(rb1, acc.at[dst_v.at[NH - 3]], add=True)
            pltpu.async_copy(t_hbm.at[src_v.at[NH - 1]], rb1, g1)
            pltpu.make_async_copy(t_hbm.at[src_v.at[NH - 2]], rb0, g0).wait()
            pltpu.sync_copy(rb0, acc.at[dst_v.at[NH - 2]], add=True)
            pltpu.make_async_copy(t_hbm.at[src_v.at[NH - 1]], rb1, g1).wait()
            pltpu.sync_copy(rb1, acc.at[dst_v.at[NH - 1]], add=True)

        plsc.subcore_barrier()

        @pl.when(s < 15)
        def _():
            pltpu.sync_copy(acc.at[pl.ds(s * RA, RA)],
                            out.at[pl.ds(c * N + s * RA, RA)])

        @pl.when(s == 15)
        def _():
            pltpu.sync_copy(acc.at[pl.ds(15 * RA, RB)],
                            out.at[pl.ds(c * N + 15 * RA, RB)])

    return run(table, src_idx, dst_idx, zeros_pad)


def _sc_degrees(eidx, vidx, ones_k, zeros_pad):
    shp = jax.ShapeDtypeStruct((NC * N, 128), jnp.float32)

    @functools.partial(
        pl.kernel,
        out_type=(shp, shp),
        mesh=_mesh,
        scratch_types=[
            pltpu.VMEM_SHARED((N, 128), jnp.float32),
            pltpu.VMEM((NCH, K), jnp.int32),
            pltpu.VMEM((K, 128), jnp.float32),
        ],
    )
    def run(e_hbm, v_hbm, ones_hbm, z_hbm, cnt_e, cnt_v,
            acc, idx_v, ones_v):
        c = lax.axis_index("c")
        s = lax.axis_index("s")
        pltpu.sync_copy(ones_hbm, ones_v)
        for idx_hbm, out in ((e_hbm, cnt_e), (v_hbm, cnt_v)):
            pltpu.sync_copy(idx_hbm.at[c * NS + s], idx_v)

            @pl.when(s < 15)
            def _():
                pltpu.sync_copy(z_hbm.at[pl.ds(s * RA, RA)],
                                acc.at[pl.ds(s * RA, RA)])

            @pl.when(s == 15)
            def _():
                pltpu.sync_copy(z_hbm.at[pl.ds(15 * RA, RB)],
                                acc.at[pl.ds(15 * RA, RB)])

            plsc.subcore_barrier()

            @pl.loop(0, NCH)
            def _(j):
                pltpu.sync_copy(ones_v, acc.at[idx_v.at[j]], add=True)

            plsc.subcore_barrier()

            @pl.when(s < 15)
            def _():
                pltpu.sync_copy(acc.at[pl.ds(s * RA, RA)],
                                out.at[pl.ds(c * N + s * RA, RA)])

            @pl.when(s == 15)
            def _():
                pltpu.sync_copy(acc.at[pl.ds(15 * RA, RB)],
                                out.at[pl.ds(c * N + 15 * RA, RB)])

            plsc.subcore_barrier()

    return run(eidx, vidx, ones_k, zeros_pad)


_BLK = 1000


def _rowspec(d):
    return pl.BlockSpec((_BLK, d), lambda i: (i, 0))


def _dot(a, b):
    return lax.dot_general(a, b, (((1,), (0,)), ((), ())),
                           preferred_element_type=jnp.float32,
                           precision=lax.Precision.HIGHEST)


def _tc_matmul(X, W, b, d_pad):
    n, d_in = X.shape
    d_out = W.shape[1]

    def body(x_ref, w_ref, b_ref, o_ref):
        y = _dot(x_ref[...], w_ref[...]) + b_ref[...]
        if d_pad > d_out:
            y = jnp.concatenate(
                [y, jnp.zeros((_BLK, d_pad - d_out), jnp.float32)], axis=1)
        o_ref[...] = y

    return pl.pallas_call(
        body,
        grid=(n // _BLK,),
        in_specs=[_rowspec(d_in),
                  pl.BlockSpec((d_in, d_out), lambda i: (0, 0)),
                  pl.BlockSpec((1, d_out), lambda i: (0, 0))],
        out_specs=_rowspec(d_pad),
        out_shape=jax.ShapeDtypeStruct((n, d_pad), jnp.float32),
    )(X, W, b.reshape(1, -1))


def _tc_inv(cnt):
    def body(c0_ref, c1_ref, o_ref):
        t = jnp.maximum(c0_ref[...][:, 0:8] + c1_ref[...][:, 0:8], 1.0)
        o_ref[...] = 1.0 / t

    return pl.pallas_call(
        body,
        grid=(N // _BLK,),
        in_specs=[_rowspec(128), _rowspec(128)],
        out_specs=_rowspec(8),
        out_shape=jax.ShapeDtypeStruct((N, 8), jnp.float32),
    )(cnt[:N], cnt[N:])


def _tc_scale(ssum, inv, relu, d_out=None):
    d = ssum.shape[1]
    d_out = d_out or d

    def body(s0_ref, s1_ref, i_ref, o_ref):
        r = (s0_ref[...][:, :d_out] + s1_ref[...][:, :d_out]) * i_ref[...][:, 0:1]
        if relu:
            r = jnp.maximum(r, 0.0)
        o_ref[...] = r

    return pl.pallas_call(
        body,
        grid=(N // _BLK,),
        in_specs=[_rowspec(d), _rowspec(d), _rowspec(8)],
        out_specs=_rowspec(d_out),
        out_shape=jax.ShapeDtypeStruct((N, d_out), jnp.float32),
    )(ssum[:N], ssum[N:], inv)


def _tc_boundary(ssum, inv, x_res, W, b, d_pad):
    d = ssum.shape[1]
    d_out = W.shape[1]
    with_res = x_res is not None

    def body(*refs):
        if with_res:
            s0_ref, s1_ref, i_ref, xr_ref, w_ref, b_ref, z_ref, y_ref = refs
        else:
            s0_ref, s1_ref, i_ref, w_ref, b_ref, z_ref, y_ref = refs
        z = jnp.maximum((s0_ref[...] + s1_ref[...]) * i_ref[...][:, 0:1], 0.0)
        if with_res:
            z = z + xr_ref[...]
        z_ref[...] = z
        y = _dot(z, w_ref[...]) + b_ref[...]
        if d_pad > d_out:
            y = jnp.concatenate(
                [y, jnp.zeros((_BLK, d_pad - d_out), jnp.float32)], axis=1)
        y_ref[...] = y

    in_specs = [_rowspec(d), _rowspec(d), _rowspec(8)]
    args = [ssum[:N], ssum[N:], inv]
    if with_res:
        in_specs.append(_rowspec(d))
        args.append(x_res)
    in_specs += [pl.BlockSpec((d, d_out), lambda i: (0, 0)),
                 pl.BlockSpec((1, d_out), lambda i: (0, 0))]
    args += [W, b.reshape(1, -1)]

    return pl.pallas_call(
        body,
        grid=(N // _BLK,),
        in_specs=in_specs,
        out_specs=[_rowspec(d), _rowspec(d_pad)],
        out_shape=[jax.ShapeDtypeStruct((N, d), jnp.float32),
                   jax.ShapeDtypeStruct((N, d_pad), jnp.float32)],
    )(*args)


def kernel(X, edge_index, W1, b1, W2, b2, W3, b3):
    vids = edge_index[0].reshape(NC * NS * NHV, NH, K)
    eids = edge_index[1].reshape(NC * NS * NHV, NH, K)
    vids_d = edge_index[0].reshape(NC * NS, NCH, K)
    eids_d = edge_index[1].reshape(NC * NS, NCH, K)
    zeros_pad = jnp.zeros((N, 128), jnp.float32)
    zeros_cnt = jnp.zeros((N, 128), jnp.float32)
    ones_k = jnp.ones((K, 128), jnp.float32)

    cnt_e, cnt_v = _sc_degrees(eids_d, vids_d, ones_k, zeros_cnt)
    inv_e = _tc_inv(cnt_e)
    inv_v = _tc_inv(cnt_v)

    y1 = _tc_matmul(X, W1, b1, 128)
    s = _sc_segment_sum(y1, vids, eids, zeros_pad)
    e1 = _tc_scale(s, inv_e, relu=False)
    s = _sc_segment_sum(e1, eids, vids, zeros_pad)
    x1, y2 = _tc_boundary(s, inv_v, None, W2, b2, 128)

    s = _sc_segment_sum(y2, vids, eids, zeros_pad)
    e2 = _tc_scale(s, inv_e, relu=False)
    s = _sc_segment_sum(e2, eids, vids, zeros_pad)
    _, y3 = _tc_boundary(s, inv_v, x1, W3, b3, 128)

    s = _sc_segment_sum(y3, vids, eids, zeros_pad)
    e3 = _tc_scale(s, inv_e, relu=False)
    s = _sc_segment_sum(e3, eids, vids, zeros_pad)
    x3 = _tc_scale(s, inv_v, relu=True, d_out=64)
    return x3
